# baseline (device time: 9724 ns/iter reference)
import jax
import jax.numpy as jnp
from jax import lax
from jax.experimental import pallas as pl
from jax.experimental.pallas import tpu as pltpu

BM = 128


def kernel(x):
    m, n = x.shape
    grid = m // BM
    n_slices = n // 128

    def body(x_ref, out_ref, send_ref, recv_ref, send_sem, recv_sem):
        i = pl.program_id(0)
        my_x = lax.axis_index("x")
        my_y = lax.axis_index("y")
        peer = (my_x, 1 - my_y)
        barrier_sem = pltpu.get_barrier_semaphore()

        @pl.when(i == 0)
        def _():
            pl.semaphore_signal(
                barrier_sem, inc=1,
                device_id=peer, device_id_type=pl.DeviceIdType.MESH,
            )

        p = x_ref[:, 0:128]
        for k in range(1, n_slices):
            p = jnp.maximum(p, x_ref[:, k * 128:(k + 1) * 128])
        colmax = jnp.max(p.T, axis=0, keepdims=True)
        send_ref[pl.ds(i, 1), :] = colmax

        @pl.when(i == grid - 1)
        def _():
            pl.semaphore_wait(barrier_sem, 1)
            rdma = pltpu.make_async_remote_copy(
                src_ref=send_ref,
                dst_ref=recv_ref,
                send_sem=send_sem,
                recv_sem=recv_sem,
                device_id=peer,
                device_id_type=pl.DeviceIdType.MESH,
            )
            rdma.start()
            rdma.wait()
            out_ref[:, :] = jnp.maximum(send_ref[:, :], recv_ref[:, :])

    out = pl.pallas_call(
        body,
        grid=(grid,),
        out_shape=jax.ShapeDtypeStruct((grid, BM), x.dtype),
        in_specs=[pl.BlockSpec((BM, n), lambda i: (i, 0))],
        out_specs=pl.BlockSpec((grid, BM), lambda i: (0, 0)),
        scratch_shapes=[
            pltpu.VMEM((grid, BM), x.dtype),
            pltpu.VMEM((grid, BM), x.dtype),
            pltpu.SemaphoreType.DMA,
            pltpu.SemaphoreType.DMA,
        ],
        compiler_params=pltpu.CompilerParams(collective_id=0),
    )(x)
    return out.reshape(m, 1)


# device time: 8934 ns/iter; 1.0884x vs baseline; 1.0884x over previous
import jax
import jax.numpy as jnp
from jax import lax
from jax.experimental import pallas as pl
from jax.experimental.pallas import tpu as pltpu

BM = 256


def kernel(x):
    m, n = x.shape
    grid = m // BM
    half = grid // 2
    n_slices = n // 128

    def body(x_ref, out_ref, send_ref, recv_ref, send_sems, recv_sems):
        i = pl.program_id(0)
        my_x = lax.axis_index("x")
        my_y = lax.axis_index("y")
        peer = (my_x, 1 - my_y)
        barrier_sem = pltpu.get_barrier_semaphore()

        @pl.when(i == 0)
        def _():
            pl.semaphore_signal(
                barrier_sem, inc=1,
                device_id=peer, device_id_type=pl.DeviceIdType.MESH,
            )

        p = x_ref[:, 0:128]
        for k in range(1, n_slices):
            p = jnp.maximum(p, x_ref[:, k * 128:(k + 1) * 128])
        send_ref[pl.ds(i, 1), :] = jnp.max(p.T, axis=0, keepdims=True)

        def push(h):
            rdma = pltpu.make_async_remote_copy(
                src_ref=send_ref.at[pl.ds(h * half, half)],
                dst_ref=recv_ref.at[pl.ds(h * half, half)],
                send_sem=send_sems.at[h],
                recv_sem=recv_sems.at[h],
                device_id=peer,
                device_id_type=pl.DeviceIdType.MESH,
            )
            rdma.start()
            return rdma

        @pl.when(i == half - 1)
        def _():
            pl.semaphore_wait(barrier_sem, 1)
            push(0)

        @pl.when(i == grid - 1)
        def _():
            rdma1 = push(1)
            rdma0 = pltpu.make_async_remote_copy(
                src_ref=send_ref.at[pl.ds(0, half)],
                dst_ref=recv_ref.at[pl.ds(0, half)],
                send_sem=send_sems.at[0],
                recv_sem=recv_sems.at[0],
                device_id=peer,
                device_id_type=pl.DeviceIdType.MESH,
            )
            rdma0.wait()
            rdma1.wait()
            out_ref[:, :] = jnp.maximum(send_ref[:, :], recv_ref[:, :])

    out = pl.pallas_call(
        body,
        grid=(grid,),
        out_shape=jax.ShapeDtypeStruct((grid, BM), x.dtype),
        in_specs=[pl.BlockSpec((BM, n), lambda i: (i, 0))],
        out_specs=pl.BlockSpec((grid, BM), lambda i: (0, 0)),
        scratch_shapes=[
            pltpu.VMEM((grid, BM), x.dtype),
            pltpu.VMEM((grid, BM), x.dtype),
            pltpu.SemaphoreType.DMA((2,)),
            pltpu.SemaphoreType.DMA((2,)),
        ],
        compiler_params=pltpu.CompilerParams(collective_id=0),
    )(x)
    return out.reshape(m, 1)


# device time: 4857 ns/iter; 2.0021x vs baseline; 1.8394x over previous
import jax
import jax.numpy as jnp
from jax import lax
from jax.experimental import pallas as pl
from jax.experimental.pallas import tpu as pltpu

BM = 512


def kernel(x):
    m, n = x.shape
    grid = m // BM
    half = grid // 2
    n_slices = n // 128

    def body(x_ref, out_ref, send_ref, recv_ref, send_sems, recv_sems):
        i = pl.program_id(0)
        my_x = lax.axis_index("x")
        my_y = lax.axis_index("y")
        peer = (my_x, 1 - my_y)
        barrier_sem = pltpu.get_barrier_semaphore()

        @pl.when(i == 0)
        def _():
            pl.semaphore_signal(
                barrier_sem, inc=1,
                device_id=peer, device_id_type=pl.DeviceIdType.MESH,
            )

        p = x_ref[:, 0:128]
        for k in range(1, n_slices):
            p = jnp.maximum(p, x_ref[:, k * 128:(k + 1) * 128])
        send_ref[pl.ds(i, 1), :] = jnp.max(p.T, axis=0, keepdims=True)

        def push(h):
            rdma = pltpu.make_async_remote_copy(
                src_ref=send_ref.at[pl.ds(h * half, half)],
                dst_ref=recv_ref.at[pl.ds(h * half, half)],
                send_sem=send_sems.at[h],
                recv_sem=recv_sems.at[h],
                device_id=peer,
                device_id_type=pl.DeviceIdType.MESH,
            )
            rdma.start()
            return rdma

        @pl.when(i == half - 1)
        def _():
            pl.semaphore_wait(barrier_sem, 1)
            push(0)

        @pl.when(i == grid - 1)
        def _():
            rdma1 = push(1)
            rdma0 = pltpu.make_async_remote_copy(
                src_ref=send_ref.at[pl.ds(0, half)],
                dst_ref=recv_ref.at[pl.ds(0, half)],
                send_sem=send_sems.at[0],
                recv_sem=recv_sems.at[0],
                device_id=peer,
                device_id_type=pl.DeviceIdType.MESH,
            )
            rdma0.wait()
            rdma1.wait()
            out_ref[:, :] = jnp.maximum(send_ref[:, :], recv_ref[:, :])

    out = pl.pallas_call(
        body,
        grid=(grid,),
        out_shape=jax.ShapeDtypeStruct((grid, BM), x.dtype),
        in_specs=[pl.BlockSpec((BM, n), lambda i: (i, 0))],
        out_specs=pl.BlockSpec((grid, BM), lambda i: (0, 0)),
        scratch_shapes=[
            pltpu.VMEM((grid, BM), x.dtype),
            pltpu.VMEM((grid, BM), x.dtype),
            pltpu.SemaphoreType.DMA((2,)),
            pltpu.SemaphoreType.DMA((2,)),
        ],
        compiler_params=pltpu.CompilerParams(collective_id=0),
    )(x)
    return out.reshape(m, 1)


# device time: 4116 ns/iter; 2.3625x vs baseline; 1.1800x over previous
import jax
import jax.numpy as jnp
from jax import lax
from jax.experimental import pallas as pl
from jax.experimental.pallas import tpu as pltpu

BM = 512


def kernel(x):
    m, n = x.shape
    grid = m // BM
    half = grid // 2
    n_slices = n // 128

    def body(x_ref, out_ref, send_ref, recv_ref, send_sems, recv_sems):
        i = pl.program_id(0)
        my_x = lax.axis_index("x")
        my_y = lax.axis_index("y")
        peer = (my_x, 1 - my_y)
        barrier_sem = pltpu.get_barrier_semaphore()

        @pl.when(i == 0)
        def _():
            pl.semaphore_signal(
                barrier_sem, inc=1,
                device_id=peer, device_id_type=pl.DeviceIdType.MESH,
            )

        parts = [x_ref[:, k * 128:(k + 1) * 128] for k in range(n_slices)]
        while len(parts) > 1:
            parts = [
                jnp.maximum(parts[j], parts[j + 1])
                for j in range(0, len(parts) - 1, 2)
            ] + ([parts[-1]] if len(parts) % 2 else [])
        send_ref[pl.ds(i, 1), :] = jnp.max(parts[0].T, axis=0, keepdims=True)

        def push(h):
            rdma = pltpu.make_async_remote_copy(
                src_ref=send_ref.at[pl.ds(h * half, half)],
                dst_ref=recv_ref.at[pl.ds(h * half, half)],
                send_sem=send_sems.at[h],
                recv_sem=recv_sems.at[h],
                device_id=peer,
                device_id_type=pl.DeviceIdType.MESH,
            )
            rdma.start()
            return rdma

        @pl.when(i == half - 1)
        def _():
            pl.semaphore_wait(barrier_sem, 1)
            push(0)

        @pl.when(i == grid - 1)
        def _():
            rdma1 = push(1)
            rdma0 = pltpu.make_async_remote_copy(
                src_ref=send_ref.at[pl.ds(0, half)],
                dst_ref=recv_ref.at[pl.ds(0, half)],
                send_sem=send_sems.at[0],
                recv_sem=recv_sems.at[0],
                device_id=peer,
                device_id_type=pl.DeviceIdType.MESH,
            )
            rdma0.wait()
            rdma1.wait()
            out_ref[:, :] = jnp.maximum(send_ref[:, :], recv_ref[:, :])

    out = pl.pallas_call(
        body,
        grid=(grid,),
        out_shape=jax.ShapeDtypeStruct((grid, BM), x.dtype),
        in_specs=[pl.BlockSpec((BM, n), lambda i: (i, 0))],
        out_specs=pl.BlockSpec((grid, BM), lambda i: (0, 0)),
        scratch_shapes=[
            pltpu.VMEM((grid, BM), x.dtype),
            pltpu.VMEM((grid, BM), x.dtype),
            pltpu.SemaphoreType.DMA((2,)),
            pltpu.SemaphoreType.DMA((2,)),
        ],
        compiler_params=pltpu.CompilerParams(collective_id=0),
    )(x)
    return out.reshape(m, 1)
